# baseline (speedup 1.0000x reference)
import functools

import jax
import jax.numpy as jnp
from jax import lax
from jax.experimental import pallas as pl
from jax.experimental.pallas import tpu as pltpu
from jax.experimental.pallas import tpu_sc as plsc

_D = 14336
_V = 128
_ROWS = 8192
_NC = 2
_NS = 16
_NW = _NC * _NS
_RPW = _ROWS // _NW
_NSEM = 16


def _sc_body(table_hbm, idx_hbm, out_hbm, idx_v, table_sp, *sems):
    sid = lax.axis_index("s")
    wid = sid * _NC + lax.axis_index("c")
    base = wid * _RPW

    rows_per_tile = _V // _NS
    pltpu.sync_copy(table_hbm.at[pl.ds(sid * rows_per_tile, rows_per_tile)],
                    table_sp.at[pl.ds(sid * rows_per_tile, rows_per_tile)])
    pltpu.sync_copy(idx_hbm.at[pl.ds(base, _RPW)], idx_v)
    plsc.subcore_barrier()

    def start(v, r, k):
        pltpu.make_async_copy(
            table_sp.at[pl.ds(v, 1)],
            out_hbm.at[pl.ds(base + r, 1)], sems[k]).start()

    def wait(k):
        pltpu.make_async_copy(
            table_sp.at[pl.ds(0, 1)],
            out_hbm.at[pl.ds(base, 1)], sems[k]).wait()

    vec0 = idx_v[pl.ds(0, _NSEM)]
    for k in range(_NSEM):
        start(vec0[k], k, k)

    def body(j, carry):
        vec = idx_v[pl.ds(_NSEM * (j + 1), _NSEM)]
        for k in range(_NSEM):
            wait(k)
            start(vec[k], _NSEM * (j + 1) + k, k)
        return carry

    lax.fori_loop(0, _RPW // _NSEM - 1, body, 0)

    for k in range(_NSEM):
        wait(k)


@functools.partial(
    pl.kernel,
    mesh=plsc.VectorSubcoreMesh(core_axis_name="c", subcore_axis_name="s"),
    out_type=jax.ShapeDtypeStruct((_ROWS, _D), jnp.float32),
    scratch_types=(
        [pltpu.VMEM((_RPW,), jnp.int32),
         pltpu.VMEM_SHARED((_V, _D), jnp.float32)]
        + [pltpu.SemaphoreType.DMA] * _NSEM
    ),
)
def _sc_gather(table_hbm, idx_hbm, out_hbm, *scratch):
    _sc_body(table_hbm, idx_hbm, out_hbm, *scratch)


@jax.jit
def kernel(prefix, table):
    b, s = prefix.shape
    idx = prefix.reshape(_ROWS).astype(jnp.int32)
    out = _sc_gather(table, idx)
    return out.reshape(b, s, _D)
